# baseline (speedup 1.0000x reference)
import jax
import jax.numpy as jnp
from jax.experimental import pallas as pl
from jax.experimental.pallas import tpu as pltpu
from jax.experimental.pallas import tpu_sc as plsc

N, K, C, EDGE = 10000, 32, 128, 16
NUM_EDGES = N * K

GATHER_WINDOW = 256
NB = 400


def _sc_gather(x2d, idx_flat, num_rows):
    mesh = plsc.VectorSubcoreMesh(core_axis_name="core", subcore_axis_name="subcore")
    idx2d = idx_flat.reshape(1, num_rows)

    @pl.kernel(
        out_type=jax.ShapeDtypeStruct((num_rows, C), jnp.float32),
        mesh=mesh,
    )
    def kern(x_hbm, i_hbm, o_hbm):
        def body(i_vmem, o_vmem):
            pltpu.sync_copy(x_hbm.at[i_vmem.at[0]], o_vmem)

        pltpu.emit_pipeline(
            body,
            grid=(num_rows // GATHER_WINDOW,),
            in_specs=[pl.BlockSpec((1, GATHER_WINDOW), lambda i: (0, i))],
            out_specs=[pl.BlockSpec((GATHER_WINDOW, C), lambda i: (i, 0))],
            core_axis_name=("core", "subcore"),
            dimension_semantics=(pltpu.PARALLEL,),
        )(i_hbm, o_hbm)

    return kern(x2d, idx2d)


def _filter_body(ef_ref, xj_ref, w1_ref, b1_ref, w2_ref, b2_ref, out_ref):
    efk = ef_ref[0]
    s1 = jax.lax.dot_general(
        efk, w1_ref[...], (((0,), (0,)), ((), ())),
        preferred_element_type=jnp.float32,
    )
    s1 = s1 + b1_ref[...]
    u = s1 + s1 * jax.lax.erf(s1)
    s2 = jnp.dot(u, w2_ref[...], preferred_element_type=jnp.float32)
    s2 = s2 + b2_ref[...]
    v = s2 + s2 * jax.lax.erf(s2)
    contrib = v * xj_ref[0]

    @pl.when(pl.program_id(0) == 0)
    def _():
        out_ref[...] = contrib

    @pl.when(pl.program_id(0) != 0)
    def _():
        out_ref[...] = out_ref[...] + contrib


def _tc_fused(ef_t, xj3, w1, b1, w2, b2, k_chunk):
    return pl.pallas_call(
        _filter_body,
        grid=(k_chunk,),
        in_specs=[
            pl.BlockSpec((1, EDGE, N), lambda k: (k, 0, 0)),
            pl.BlockSpec((1, N, C), lambda k: (k, 0, 0)),
            pl.BlockSpec((EDGE, C), lambda k: (0, 0)),
            pl.BlockSpec((1, C), lambda k: (0, 0)),
            pl.BlockSpec((C, C), lambda k: (0, 0)),
            pl.BlockSpec((1, C), lambda k: (0, 0)),
        ],
        out_specs=pl.BlockSpec((N, C), lambda k: (0, 0)),
        out_shape=jax.ShapeDtypeStruct((N, C), jnp.float32),
    )(ef_t, xj3, w1, b1, w2, b2)


_INV_SQRT2 = 0.7071067811865476


K_CHUNKS = 2


def kernel(x, edge_features, E_idx, W1, b1, W2, b2):
    x2d = x[0] * _INV_SQRT2
    idx_km = jnp.transpose(E_idx[0], (1, 0)).astype(jnp.int32)
    ef_t = jnp.transpose(edge_features[0], (1, 2, 0))
    w1s = W1 * _INV_SQRT2
    b1s = b1.reshape(1, C) * _INV_SQRT2
    w2s = W2 * 0.5
    b2s = b2.reshape(1, C) * _INV_SQRT2
    kc = K // K_CHUNKS
    rows = kc * N
    parts = []
    for c in range(K_CHUNKS):
        idx_c = jax.lax.slice_in_dim(idx_km, c * kc, (c + 1) * kc, axis=0)
        ef_c = jax.lax.slice_in_dim(ef_t, c * kc, (c + 1) * kc, axis=0)
        xj = _sc_gather(x2d, idx_c.reshape(rows), rows)
        parts.append(_tc_fused(ef_c, xj.reshape(kc, N, C), w1s, b1s, w2s, b2s, kc))
    out = parts[0]
    for p in parts[1:]:
        out = out + p
    return out.reshape(1, N, C)


# speedup vs baseline: 1.2813x; 1.2813x over previous
import jax
import jax.numpy as jnp
from jax.experimental import pallas as pl
from jax.experimental.pallas import tpu as pltpu
from jax.experimental.pallas import tpu_sc as plsc

N, K, C, EDGE = 10000, 32, 128, 16
NUM_EDGES = N * K

GATHER_WINDOW = 128
NB = 400


def _sc_gather(x2d, idx_flat, num_rows):
    from jax import lax

    mesh = plsc.VectorSubcoreMesh(core_axis_name="core", subcore_axis_name="subcore")
    idx2d = idx_flat.reshape(1, num_rows)

    @pl.kernel(
        out_type=jax.ShapeDtypeStruct((num_rows, C), jnp.float32),
        mesh=mesh,
        scratch_types=[
            pltpu.VMEM_SHARED((N, C), jnp.float32),
        ],
    )
    def kern(x_hbm, i_hbm, o_hbm, x_sp):
        @pl.when(lax.axis_index("subcore") == 0)
        def _():
            pltpu.sync_copy(x_hbm, x_sp)

        plsc.subcore_barrier()

        def body(i_vmem, o_vmem):
            pltpu.sync_copy(x_sp.at[i_vmem.at[0]], o_vmem)

        pltpu.emit_pipeline(
            body,
            grid=(num_rows // GATHER_WINDOW,),
            in_specs=[pl.BlockSpec((1, GATHER_WINDOW), lambda i: (0, i))],
            out_specs=[pl.BlockSpec((GATHER_WINDOW, C), lambda i: (i, 0))],
            core_axis_name=("core", "subcore"),
            dimension_semantics=(pltpu.PARALLEL,),
        )(i_hbm, o_hbm)

    return kern(x2d, idx2d)


def _filter_body(ef_ref, xj_ref, w1_ref, b1_ref, w2_ref, b2_ref, out_ref):
    efk = ef_ref[0]
    s1 = jax.lax.dot_general(
        efk, w1_ref[...], (((0,), (0,)), ((), ())),
        preferred_element_type=jnp.float32,
    )
    s1 = s1 + b1_ref[...]
    u = s1 + s1 * jax.lax.erf(s1)
    s2 = jnp.dot(u, w2_ref[...], preferred_element_type=jnp.float32)
    s2 = s2 + b2_ref[...]
    v = s2 + s2 * jax.lax.erf(s2)
    contrib = v * xj_ref[0]

    @pl.when(pl.program_id(0) == 0)
    def _():
        out_ref[...] = contrib

    @pl.when(pl.program_id(0) != 0)
    def _():
        out_ref[...] = out_ref[...] + contrib


def _tc_fused(ef_t, xj3, w1, b1, w2, b2, k_chunk):
    return pl.pallas_call(
        _filter_body,
        grid=(k_chunk,),
        in_specs=[
            pl.BlockSpec((1, EDGE, N), lambda k: (k, 0, 0)),
            pl.BlockSpec((1, N, C), lambda k: (k, 0, 0)),
            pl.BlockSpec((EDGE, C), lambda k: (0, 0)),
            pl.BlockSpec((1, C), lambda k: (0, 0)),
            pl.BlockSpec((C, C), lambda k: (0, 0)),
            pl.BlockSpec((1, C), lambda k: (0, 0)),
        ],
        out_specs=pl.BlockSpec((N, C), lambda k: (0, 0)),
        out_shape=jax.ShapeDtypeStruct((N, C), jnp.float32),
    )(ef_t, xj3, w1, b1, w2, b2)


_INV_SQRT2 = 0.7071067811865476


K_CHUNKS = 2


def kernel(x, edge_features, E_idx, W1, b1, W2, b2):
    x2d = x[0] * _INV_SQRT2
    idx_km = jnp.transpose(E_idx[0], (1, 0)).astype(jnp.int32)
    ef_t = jnp.transpose(edge_features[0], (1, 2, 0))
    w1s = W1 * _INV_SQRT2
    b1s = b1.reshape(1, C) * _INV_SQRT2
    w2s = W2 * 0.5
    b2s = b2.reshape(1, C) * _INV_SQRT2
    kc = K // K_CHUNKS
    rows = kc * N
    parts = []
    for c in range(K_CHUNKS):
        idx_c = jax.lax.slice_in_dim(idx_km, c * kc, (c + 1) * kc, axis=0)
        ef_c = jax.lax.slice_in_dim(ef_t, c * kc, (c + 1) * kc, axis=0)
        xj = _sc_gather(x2d, idx_c.reshape(rows), rows)
        parts.append(_tc_fused(ef_c, xj.reshape(kc, N, C), w1s, b1s, w2s, b2s, kc))
    out = parts[0]
    for p in parts[1:]:
        out = out + p
    return out.reshape(1, N, C)


# speedup vs baseline: 1.3180x; 1.0286x over previous
import jax
import jax.numpy as jnp
from jax.experimental import pallas as pl
from jax.experimental.pallas import tpu as pltpu
from jax.experimental.pallas import tpu_sc as plsc

N, K, C, EDGE = 10000, 32, 128, 16
NUM_EDGES = N * K

GATHER_WINDOW = 128
NB = 400


def _sc_gather(x2d, idx2d, num_rows, step_base):
    from jax import lax

    mesh = plsc.VectorSubcoreMesh(core_axis_name="core", subcore_axis_name="subcore")

    @pl.kernel(
        out_type=jax.ShapeDtypeStruct((num_rows, C), jnp.float32),
        mesh=mesh,
        scratch_types=[
            pltpu.VMEM_SHARED((N, C), jnp.float32),
        ],
    )
    def kern(x_hbm, i_hbm, o_hbm, x_sp):
        @pl.when(lax.axis_index("subcore") == 0)
        def _():
            pltpu.sync_copy(x_hbm, x_sp)

        plsc.subcore_barrier()

        def body(i_vmem, o_vmem):
            pltpu.sync_copy(x_sp.at[i_vmem.at[0]], o_vmem)

        pltpu.emit_pipeline(
            body,
            grid=(num_rows // GATHER_WINDOW,),
            in_specs=[pl.BlockSpec((1, GATHER_WINDOW), lambda i: (0, step_base + i))],
            out_specs=[pl.BlockSpec((GATHER_WINDOW, C), lambda i: (i, 0))],
            core_axis_name=("core", "subcore"),
            dimension_semantics=(pltpu.PARALLEL,),
        )(i_hbm, o_hbm)

    return kern(x2d, idx2d)


def _filter_body(ef_ref, xj_ref, w1_ref, b1_ref, w2_ref, b2_ref, out_ref):
    efk = ef_ref[0]
    s1 = jax.lax.dot_general(
        efk, w1_ref[...], (((0,), (0,)), ((), ())),
        preferred_element_type=jnp.float32,
    )
    s1 = s1 + b1_ref[...]
    u = s1 + s1 * jax.lax.erf(s1)
    s2 = jnp.dot(u, w2_ref[...], preferred_element_type=jnp.float32)
    s2 = s2 + b2_ref[...]
    v = s2 + s2 * jax.lax.erf(s2)
    contrib = v * xj_ref[0]

    @pl.when(pl.program_id(0) == 0)
    def _():
        out_ref[...] = contrib

    @pl.when(pl.program_id(0) != 0)
    def _():
        out_ref[...] = out_ref[...] + contrib


def _tc_fused(ef_t, xj3, w1, b1, w2, b2, k_chunk, k_base):
    return pl.pallas_call(
        _filter_body,
        grid=(k_chunk,),
        in_specs=[
            pl.BlockSpec((1, EDGE, N), lambda k: (k_base + k, 0, 0)),
            pl.BlockSpec((1, N, C), lambda k: (k, 0, 0)),
            pl.BlockSpec((EDGE, C), lambda k: (0, 0)),
            pl.BlockSpec((1, C), lambda k: (0, 0)),
            pl.BlockSpec((C, C), lambda k: (0, 0)),
            pl.BlockSpec((1, C), lambda k: (0, 0)),
        ],
        out_specs=pl.BlockSpec((N, C), lambda k: (0, 0)),
        out_shape=jax.ShapeDtypeStruct((N, C), jnp.float32),
    )(ef_t, xj3, w1, b1, w2, b2)


_INV_SQRT2 = 0.7071067811865476


K_CHUNKS = 2


def kernel(x, edge_features, E_idx, W1, b1, W2, b2):
    x2d = x[0] * _INV_SQRT2
    idx_km = jnp.transpose(E_idx[0], (1, 0)).astype(jnp.int32)
    ef_t = jnp.transpose(edge_features[0], (1, 2, 0))
    w1s = W1 * _INV_SQRT2
    b1s = b1.reshape(1, C) * _INV_SQRT2
    w2s = W2 * 0.5
    b2s = b2.reshape(1, C) * _INV_SQRT2
    kc = K // K_CHUNKS
    rows = kc * N
    idx2d = idx_km.reshape(1, NUM_EDGES)
    parts = []
    for c in range(K_CHUNKS):
        xj = _sc_gather(x2d, idx2d, rows, c * (rows // GATHER_WINDOW))
        parts.append(
            _tc_fused(ef_t, xj.reshape(kc, N, C), w1s, b1s, w2s, b2s, kc, c * kc)
        )
    out = parts[0]
    for p in parts[1:]:
        out = out + p
    return out.reshape(1, N, C)


# speedup vs baseline: 1.3471x; 1.0221x over previous
import jax
import jax.numpy as jnp
from jax.experimental import pallas as pl
from jax.experimental.pallas import tpu as pltpu
from jax.experimental.pallas import tpu_sc as plsc

N, K, C, EDGE = 10000, 32, 128, 16
NUM_EDGES = N * K

GATHER_WINDOW = 128
NB = 400


def _sc_gather(x2d, idx2d, num_rows, step_base):
    from jax import lax

    mesh = plsc.VectorSubcoreMesh(core_axis_name="core", subcore_axis_name="subcore")

    @pl.kernel(
        out_type=jax.ShapeDtypeStruct((num_rows, C), jnp.float32),
        mesh=mesh,
        scratch_types=[
            pltpu.VMEM_SHARED((N, C), jnp.float32),
        ],
    )
    def kern(x_hbm, i_hbm, o_hbm, x_sp):
        @pl.when(lax.axis_index("subcore") == 0)
        def _():
            pltpu.sync_copy(x_hbm, x_sp)

        plsc.subcore_barrier()

        def body(i_vmem, o_vmem):
            pltpu.sync_copy(x_sp.at[i_vmem.at[0]], o_vmem)

        pltpu.emit_pipeline(
            body,
            grid=(num_rows // GATHER_WINDOW,),
            in_specs=[pl.BlockSpec((1, GATHER_WINDOW), lambda i: (0, step_base + i))],
            out_specs=[pl.BlockSpec((GATHER_WINDOW, C), lambda i: (i, 0))],
            core_axis_name=("core", "subcore"),
            dimension_semantics=(pltpu.PARALLEL,),
        )(i_hbm, o_hbm)

    return kern(x2d, idx2d)


def _filter_body(ef_ref, xj_ref, w1_ref, b1_ref, w2_ref, b2_ref, out_ref):
    efk = ef_ref[0]
    s1 = jax.lax.dot_general(
        efk, w1_ref[...], (((0,), (0,)), ((), ())),
        preferred_element_type=jnp.float32,
    )
    s1 = s1 + b1_ref[...]
    u = s1 + s1 * jax.lax.erf(s1)
    s2 = jnp.dot(u, w2_ref[...], preferred_element_type=jnp.float32)
    s2 = s2 + b2_ref[...]
    v = s2 + s2 * jax.lax.erf(s2)
    contrib = v * xj_ref[0]

    @pl.when(pl.program_id(0) == 0)
    def _():
        out_ref[...] = contrib

    @pl.when(pl.program_id(0) != 0)
    def _():
        out_ref[...] = out_ref[...] + contrib


def _tc_fused(ef_t, xj3, w1, b1, w2, b2, k_chunk, k_base):
    return pl.pallas_call(
        _filter_body,
        grid=(k_chunk,),
        in_specs=[
            pl.BlockSpec((1, EDGE, N), lambda k: (k_base + k, 0, 0)),
            pl.BlockSpec((1, N, C), lambda k: (k, 0, 0)),
            pl.BlockSpec((EDGE, C), lambda k: (0, 0)),
            pl.BlockSpec((1, C), lambda k: (0, 0)),
            pl.BlockSpec((C, C), lambda k: (0, 0)),
            pl.BlockSpec((1, C), lambda k: (0, 0)),
        ],
        out_specs=pl.BlockSpec((N, C), lambda k: (0, 0)),
        out_shape=jax.ShapeDtypeStruct((N, C), jnp.float32),
    )(ef_t, xj3, w1, b1, w2, b2)


_INV_SQRT2 = 0.7071067811865476


K_CHUNKS = 2


def kernel(x, edge_features, E_idx, W1, b1, W2, b2):
    x2d = x[0]
    idx_km = jnp.transpose(E_idx[0], (1, 0)).astype(jnp.int32)
    ef_t = jnp.transpose(edge_features[0], (1, 2, 0))
    w1s = W1 * _INV_SQRT2
    b1s = b1.reshape(1, C) * _INV_SQRT2
    w2s = W2 * 0.5
    b2s = b2.reshape(1, C) * _INV_SQRT2
    kc = K // K_CHUNKS
    rows = kc * N
    idx2d = idx_km.reshape(1, NUM_EDGES)
    parts = []
    for c in range(K_CHUNKS):
        xj = _sc_gather(x2d, idx2d, rows, c * (rows // GATHER_WINDOW))
        parts.append(
            _tc_fused(ef_t, xj.reshape(kc, N, C), w1s, b1s, w2s, b2s, kc, c * kc)
        )
    out = parts[0]
    for p in parts[1:]:
        out = out + p
    out = out * _INV_SQRT2
    return out.reshape(1, N, C)


# speedup vs baseline: 1.3762x; 1.0216x over previous
import jax
import jax.numpy as jnp
from jax.experimental import pallas as pl
from jax.experimental.pallas import tpu as pltpu
from jax.experimental.pallas import tpu_sc as plsc

N, K, C, EDGE = 10000, 32, 128, 16
NUM_EDGES = N * K

GATHER_WINDOW = 128
NB = 400


def _sc_gather(x2d, idx2d, num_rows, step_base):
    from jax import lax

    mesh = plsc.VectorSubcoreMesh(core_axis_name="core", subcore_axis_name="subcore")

    @pl.kernel(
        out_type=jax.ShapeDtypeStruct((num_rows, C), jnp.float32),
        mesh=mesh,
        scratch_types=[
            pltpu.VMEM_SHARED((N, C), jnp.float32),
        ],
    )
    def kern(x_hbm, i_hbm, o_hbm, x_sp):
        @pl.when(lax.axis_index("subcore") == 0)
        def _():
            pltpu.sync_copy(x_hbm, x_sp)

        plsc.subcore_barrier()

        def body(i_vmem, o_vmem):
            pltpu.sync_copy(x_sp.at[i_vmem.at[0]], o_vmem)

        pltpu.emit_pipeline(
            body,
            grid=(num_rows // GATHER_WINDOW,),
            in_specs=[pl.BlockSpec((1, GATHER_WINDOW), lambda i: (0, step_base + i))],
            out_specs=[pl.BlockSpec((GATHER_WINDOW, C), lambda i: (i, 0))],
            core_axis_name=("core", "subcore"),
            dimension_semantics=(pltpu.PARALLEL,),
        )(i_hbm, o_hbm)

    return kern(x2d, idx2d)


def _filter_slab(ef_ref, xj_ref, w1_ref, b1_ref, w2_ref, b2_ref):
    efk = ef_ref[0]
    s1 = jax.lax.dot_general(
        efk, w1_ref[...], (((0,), (0,)), ((), ())),
        preferred_element_type=jnp.float32,
    )
    s1 = s1 + b1_ref[...]
    u = s1 + s1 * jax.lax.erf(s1)
    s2 = jnp.dot(u, w2_ref[...], preferred_element_type=jnp.float32)
    s2 = s2 + b2_ref[...]
    v = s2 + s2 * jax.lax.erf(s2)
    return v * xj_ref[0]


def _filter_body(ef_ref, xj_ref, w1_ref, b1_ref, w2_ref, b2_ref, out_ref):
    contrib = _filter_slab(ef_ref, xj_ref, w1_ref, b1_ref, w2_ref, b2_ref)

    @pl.when(pl.program_id(0) == 0)
    def _():
        out_ref[...] = contrib

    @pl.when(pl.program_id(0) != 0)
    def _():
        out_ref[...] = out_ref[...] + contrib


def _filter_body_final(ef_ref, xj_ref, w1_ref, b1_ref, w2_ref, b2_ref,
                       prev_ref, out_ref):
    contrib = _filter_slab(ef_ref, xj_ref, w1_ref, b1_ref, w2_ref, b2_ref)

    @pl.when(pl.program_id(0) == 0)
    def _():
        out_ref[...] = prev_ref[...] + contrib

    @pl.when(
        (pl.program_id(0) != 0) & (pl.program_id(0) != pl.num_programs(0) - 1)
    )
    def _():
        out_ref[...] = out_ref[...] + contrib

    @pl.when(pl.program_id(0) == pl.num_programs(0) - 1)
    def _():
        out_ref[...] = (out_ref[...] + contrib) * _INV_SQRT2


def _tc_fused(ef_t, xj3, w1, b1, w2, b2, k_chunk, k_base, prev=None):
    in_specs = [
        pl.BlockSpec((1, EDGE, N), lambda k: (k_base + k, 0, 0)),
        pl.BlockSpec((1, N, C), lambda k: (k, 0, 0)),
        pl.BlockSpec((EDGE, C), lambda k: (0, 0)),
        pl.BlockSpec((1, C), lambda k: (0, 0)),
        pl.BlockSpec((C, C), lambda k: (0, 0)),
        pl.BlockSpec((1, C), lambda k: (0, 0)),
    ]
    args = (ef_t, xj3, w1, b1, w2, b2)
    body = _filter_body
    if prev is not None:
        in_specs.append(pl.BlockSpec((N, C), lambda k: (0, 0)))
        args = args + (prev,)
        body = _filter_body_final
    return pl.pallas_call(
        body,
        grid=(k_chunk,),
        in_specs=in_specs,
        out_specs=pl.BlockSpec((N, C), lambda k: (0, 0)),
        out_shape=jax.ShapeDtypeStruct((N, C), jnp.float32),
    )(*args)


_INV_SQRT2 = 0.7071067811865476


K_CHUNKS = 2


def kernel(x, edge_features, E_idx, W1, b1, W2, b2):
    x2d = x[0]
    idx_km = jnp.transpose(E_idx[0], (1, 0)).astype(jnp.int32)
    ef_t = jnp.transpose(edge_features[0], (1, 2, 0))
    w1s = W1 * _INV_SQRT2
    b1s = b1.reshape(1, C) * _INV_SQRT2
    w2s = W2 * 0.5
    b2s = b2.reshape(1, C) * _INV_SQRT2
    kc = K // K_CHUNKS
    rows = kc * N
    idx2d = idx_km.reshape(1, NUM_EDGES)
    out = None
    for c in range(K_CHUNKS):
        xj = _sc_gather(x2d, idx2d, rows, c * (rows // GATHER_WINDOW))
        prev = out if c == K_CHUNKS - 1 else None
        out = _tc_fused(
            ef_t, xj.reshape(kc, N, C), w1s, b1s, w2s, b2s, kc, c * kc, prev=prev
        )
    return out.reshape(1, N, C)
